# Initial kernel scaffold; baseline (speedup 1.0000x reference)
import jax
import jax.numpy as jnp
from jax.experimental import pallas as pl


def kernel(x_0, x_1, x_2, up_idx, up_val, dn_idx, dn_val, inc_idx, inc_val, x_bel_1, W_in0, b_in0, W_in1, b_in1, W_in2, b_in2, W_up, a_up, W_dn, a_dn, W_har, W_out, b_out, W_ro, b_ro):
    raise NotImplementedError("write your pallas kernel here")


# speedup vs baseline: 1.0205x; 1.0205x over previous
import jax
import jax.numpy as jnp
from jax.experimental import pallas as pl

N0, N1, N2 = 25000, 50000, 10000
DIN, DB, DH, DOUT = 128, 64, 64, 64
NF, ORDER, EPS, NG = 2, 5, 0.1, 64


def _spmm(idx, val, x, n_rows):
    return jnp.zeros((n_rows, x.shape[1]), x.dtype).at[idx[0]].add(
        val[:, None] * jnp.take(x, idx[1], axis=0))


def _alpha(idx, h, a, n):
    d = h.shape[1]
    s0 = h @ a[:d]
    s1 = h @ a[d:]
    e = jax.nn.leaky_relu(s0[idx[0]] + s1[idx[1]], 0.2)
    ex = jnp.exp(e)
    s = jax.ops.segment_sum(ex, idx[0], num_segments=n)
    return ex / (s[idx[0]] + 1e-9)


def kernel(x_0, x_1, x_2, up_idx, up_val, dn_idx, dn_val, inc_idx, inc_val, x_bel_1,
           W_in0, b_in0, W_in1, b_in1, W_in2, b_in2,
           W_up, a_up, W_dn, a_dn, W_har, W_out, b_out, W_ro, b_ro):
    x = x_1 @ W_in1 + b_in1

    def san(idx, val, Ws, a):
        h = x @ Ws[0]
        alpha = _alpha(idx, h, a, N1)
        w = alpha * val
        y1 = x @ Ws[1]
        inner = h + _spmm(idx, w, y1, N1)
        return _spmm(idx, w, inner, N1)

    z_up = san(up_idx, up_val, W_up, a_up)
    z_dn = san(dn_idx, dn_val, W_dn, a_dn)

    y = x @ W_har
    for _ in range(ORDER):
        y = y - EPS * (_spmm(up_idx, up_val, y, N1) + _spmm(dn_idx, dn_val, y, N1))

    xo = jax.nn.relu(z_up + z_dn + y)
    x1_out = xo @ W_out + b_out
    pooled = jax.ops.segment_sum(x1_out, x_bel_1, num_segments=NG)
    return pooled @ W_ro + b_ro


# speedup vs baseline: 8.0926x; 7.9300x over previous
import functools

import jax
import jax.numpy as jnp
from jax import lax
from jax.experimental import pallas as pl
from jax.experimental.pallas import tpu as pltpu
from jax.experimental.pallas import tpu_sc as plsc

N1 = 50000
DIN, DB, DH, DOUT, NG = 128, 64, 64, 64, 64
DHH = DH // 2
NNZ = 800000
EPS = 0.1
ORDER = 5
NT = 16
SUB = 80
CB = 25
RT = N1 // NT
SPAD = 16 * 3136
RB = 2000
NBLK = N1 // RB

_MESH = plsc.VectorSubcoreMesh(core_axis_name="c", subcore_axis_name="s")
_SC_PARAMS = pltpu.CompilerParams(use_tc_tiling_on_sc=False,
                                  needs_layout_passes=False)


def _proj_body(x1_ref, Win_ref, bin_ref, Wcat_ref, Aup_ref, Adn_ref,
               hup, yup1, hdn, ydn1, yhar, sup_ref, sdn_ref):
    t = x1_ref[...] @ Win_ref[...] + bin_ref[...]
    p = t @ Wcat_ref[...]

    def split(m):
        return jnp.stack([m[:, :DHH], m[:, DHH:]])

    hup[...] = split(p[:, 0:64])
    yup1[...] = split(p[:, 64:128])
    hdn[...] = split(p[:, 128:192])
    ydn1[...] = split(p[:, 192:256])
    yhar[...] = split(p[:, 256:320])
    sup_ref[...] = p[:, 0:64] @ Aup_ref[...]
    sdn_ref[...] = p[:, 128:192] @ Adn_ref[...]


def _proj(x_1, W_in1, b_in1, Wcat, Aup, Adn):
    half = jax.ShapeDtypeStruct((2, N1, DHH), jnp.float32)
    svec = jax.ShapeDtypeStruct((N1, 2), jnp.float32)
    full_spec = lambda s: pl.BlockSpec(s, lambda i: (0, 0))
    return pl.pallas_call(
        _proj_body,
        grid=(NBLK,),
        in_specs=[pl.BlockSpec((RB, DIN), lambda i: (i, 0)),
                  full_spec((DIN, DB)), full_spec((1, DB)),
                  full_spec((DB, 320)), full_spec((DB, 2)), full_spec((DB, 2))],
        out_specs=[pl.BlockSpec((2, RB, DHH), lambda i: (0, i, 0))] * 5 +
                  [pl.BlockSpec((RB, 2), lambda i: (i, 0))] * 2,
        out_shape=[half] * 5 + [svec] * 2,
    )(x_1, W_in1, b_in1, Wcat, Aup, Adn)


_CRA = NNZ // SUB // NT


def _attn_body(su_cat, r_cat, c_cat, v_cat, zeros_hbm,
               w_out, sv_out,
               su_buf, rb, cb, vb, wb, svb, tb, ssb, ssum):
    cid = lax.axis_index("c")
    sid = lax.axis_index("s")
    r_hbm = r_cat.at[cid]
    c_hbm = c_cat.at[cid]
    v_hbm = v_cat.at[cid]
    w_hbm = w_out.at[cid]
    sv_hbm = sv_out.at[cid]

    def compute_t(k):
        def vec(j, _):
            sl = pl.ds(j * 16, 16)
            rv = rb[k, sl]
            cv = cb[k, sl]
            a = plsc.load_gather(su_buf, [rv * 2])
            b = plsc.load_gather(su_buf, [cv * 2 + 1])
            e = a + b
            e = jnp.where(e >= 0, e, e * jnp.float32(0.2))
            tb[sl] = jnp.exp(e)
            return 0
        lax.fori_loop(0, SUB // 16, vec, 0)

    pltpu.sync_copy(zeros_hbm.at[pl.ds(0, SPAD // NT)],
                    ssum.at[pl.ds(sid * (SPAD // NT), SPAD // NT)])
    pltpu.sync_copy(su_cat.at[cid], su_buf)
    plsc.subcore_barrier()
    cr0 = sid * _CRA

    def outer1(ob, _):
        cbase = cr0 + ob * CB
        pltpu.sync_copy(r_hbm.at[pl.ds(cbase, CB)], rb)
        pltpu.sync_copy(c_hbm.at[pl.ds(cbase, CB)], cb)

        def chunk(k, _):
            compute_t(k)
            pltpu.sync_copy(tb, ssum.at[rb.at[k]], add=True)
            return 0
        lax.fori_loop(0, CB, chunk, 0)
        return 0
    lax.fori_loop(0, _CRA // CB, outer1, 0)
    plsc.subcore_barrier()

    def outer2(ob, _):
        cbase = cr0 + ob * CB
        pltpu.sync_copy(r_hbm.at[pl.ds(cbase, CB)], rb)
        pltpu.sync_copy(c_hbm.at[pl.ds(cbase, CB)], cb)
        pltpu.sync_copy(v_hbm.at[pl.ds(cbase, CB)], vb)

        def chunk(k, _):
            compute_t(k)
            pltpu.sync_copy(ssum.at[rb.at[k]], ssb)

            def vec2(j, _):
                sl = pl.ds(j * 16, 16)
                v = vb[k, sl]
                wb[k, sl] = tb[sl] / (ssb[sl] + jnp.float32(1e-9)) * v
                svb[k, sl] = v * jnp.float32(-EPS)
                return 0
            lax.fori_loop(0, SUB // 16, vec2, 0)
            return 0
        lax.fori_loop(0, CB, chunk, 0)
        pltpu.sync_copy(wb, w_hbm.at[pl.ds(cbase, CB)])
        pltpu.sync_copy(svb, sv_hbm.at[pl.ds(cbase, CB)])
        return 0
    lax.fori_loop(0, _CRA // CB, outer2, 0)


_attn = functools.partial(
    pl.kernel,
    mesh=_MESH,
    compiler_params=_SC_PARAMS,
    out_type=[jax.ShapeDtypeStruct((2, NNZ // SUB, SUB), jnp.float32)] * 2,
    scratch_types=[
        pltpu.VMEM((2 * N1,), jnp.float32),
        pltpu.VMEM((CB, SUB), jnp.int32),
        pltpu.VMEM((CB, SUB), jnp.int32),
        pltpu.VMEM((CB, SUB), jnp.float32),
        pltpu.VMEM((CB, SUB), jnp.float32),
        pltpu.VMEM((CB, SUB), jnp.float32),
        pltpu.VMEM((SUB,), jnp.float32),
        pltpu.VMEM((SUB,), jnp.float32),
        pltpu.VMEM_SHARED((SPAD,), jnp.float32),
    ],
)(_attn_body)


def _spmm_body(nchunks, init_h, y_h, r_hbm, c_cat, w_hbm, z_h,
               rb, cb, wb, G0, G1, sg0, sg1, ss0, ss1, zacc):
    cid = lax.axis_index("c")
    sid = lax.axis_index("s")
    cr_t = nchunks // NT
    base = cid * N1 + sid * RT

    pltpu.sync_copy(init_h.at[pl.ds(base, RT)], zacc.at[pl.ds(sid * RT, RT)])
    plsc.subcore_barrier()
    cr0 = sid * cr_t
    c_hbm = c_cat.at[cid]
    Gs = (G0, G1)
    sg = (sg0, sg1)
    ss = (ss0, ss1)

    def scale(Gr, k):
        def grp(g, _):
            wv = wb[k, pl.ds(g * 16, 16)]
            gbase = g * 16
            for e in range(16):
                row = gbase + e
                w = jnp.broadcast_to(wv[e], (16,))
                Gr[row, pl.ds(0, 16)] = Gr[row, pl.ds(0, 16)] * w
                Gr[row, pl.ds(16, 16)] = Gr[row, pl.ds(16, 16)] * w
            return 0
        lax.fori_loop(0, SUB // 16, grp, 0)

    def outer(ob, _):
        cbase = cr0 + ob * CB
        pltpu.sync_copy(r_hbm.at[pl.ds(cbase, CB)], rb)
        pltpu.sync_copy(c_hbm.at[pl.ds(cbase, CB)], cb)
        pltpu.sync_copy(w_hbm.at[pl.ds(cbase, CB)], wb)
        pltpu.async_copy(y_h.at[cb.at[0]], Gs[0], sg[0])
        for k in range(CB):
            b = k & 1
            pltpu.make_async_copy(y_h.at[cb.at[k]], Gs[b], sg[b]).wait()
            scale(Gs[b], k)
            if k >= 1:
                pltpu.make_async_copy(
                    Gs[1 - b], zacc.at[rb.at[k - 1]], ss[1 - b]).wait()
            if k + 1 < CB:
                pltpu.async_copy(y_h.at[cb.at[k + 1]], Gs[1 - b], sg[1 - b])
            pltpu.async_copy(Gs[b], zacc.at[rb.at[k]], ss[b], add=True)
        bl = (CB - 1) & 1
        pltpu.make_async_copy(Gs[bl], zacc.at[rb.at[CB - 1]], ss[bl]).wait()
        return 0
    lax.fori_loop(0, cr_t // CB, outer, 0)
    plsc.subcore_barrier()
    pltpu.sync_copy(zacc.at[pl.ds(sid * RT, RT)], z_h.at[pl.ds(base, RT)])


def _make_spmm(nchunks):
    return functools.partial(
        pl.kernel,
        mesh=_MESH,
        compiler_params=_SC_PARAMS,
        out_type=jax.ShapeDtypeStruct((2 * N1, DHH), jnp.float32),
        scratch_types=[
            pltpu.VMEM((CB, SUB), jnp.int32),
            pltpu.VMEM((CB, SUB), jnp.int32),
            pltpu.VMEM((CB, SUB), jnp.float32),
            pltpu.VMEM((SUB, DHH), jnp.float32),
            pltpu.VMEM((SUB, DHH), jnp.float32),
            pltpu.SemaphoreType.DMA,
            pltpu.SemaphoreType.DMA,
            pltpu.SemaphoreType.DMA,
            pltpu.SemaphoreType.DMA,
            pltpu.VMEM_SHARED((N1, DHH), jnp.float32),
        ],
    )(functools.partial(_spmm_body, nchunks))


_spmm8 = _make_spmm(NNZ // SUB)
_spmm16 = _make_spmm(2 * NNZ // SUB)


def _ro_body(z_ref, gid_ref, Wout_ref, bout_ref, Wro_ref, bro_ref,
             out_ref, acc_ref):
    i = pl.program_id(0)

    @pl.when(i == 0)
    def _():
        acc_ref[...] = jnp.zeros((NG, DOUT), jnp.float32)

    z = jnp.concatenate([z_ref[0], z_ref[1]], axis=1)
    x = jnp.maximum(z, 0.0) @ Wout_ref[...] + bout_ref[...]
    g = gid_ref[0, 0, :]
    oh = (g[:, None] == lax.broadcasted_iota(jnp.int32, (RB, NG), 1))
    oh = oh.astype(jnp.float32)
    acc_ref[...] += lax.dot_general(oh, x, (((0,), (0,)), ((), ())))

    @pl.when(i == pl.num_programs(0) - 1)
    def _():
        out_ref[...] = acc_ref[...] @ Wro_ref[...] + bro_ref[...]


def _readout(z2, gid3, W_out, b_out, W_ro, b_ro):
    full_spec = lambda s: pl.BlockSpec(s, lambda i: tuple(0 for _ in s))
    return pl.pallas_call(
        _ro_body,
        grid=(NBLK,),
        in_specs=[pl.BlockSpec((2, RB, DHH), lambda i: (0, i, 0)),
                  pl.BlockSpec((1, 1, RB), lambda i: (i, 0, 0)),
                  full_spec((DH, DOUT)), full_spec((1, DOUT)),
                  full_spec((DOUT, DOUT)), full_spec((1, DOUT))],
        out_specs=full_spec((NG, DOUT)),
        out_shape=jax.ShapeDtypeStruct((NG, DOUT), jnp.float32),
        scratch_shapes=[pltpu.VMEM((NG, DOUT), jnp.float32)],
    )(z2, gid3, W_out, b_out, W_ro, b_ro)


def kernel(x_0, x_1, x_2, up_idx, up_val, dn_idx, dn_val, inc_idx, inc_val,
           x_bel_1, W_in0, b_in0, W_in1, b_in1, W_in2, b_in2,
           W_up, a_up, W_dn, a_dn, W_har, W_out, b_out, W_ro, b_ro):
    f32 = jnp.float32
    NB8 = NNZ // SUB
    Wcat = jnp.concatenate([W_up[0], W_up[1], W_dn[0], W_dn[1], W_har], axis=1)
    Aup = a_up.reshape(2, DH).T
    Adn = a_dn.reshape(2, DH).T

    projs = _proj(x_1, W_in1, b_in1.reshape(1, DB), Wcat, Aup, Adn)
    hup, yup1, hdn, ydn1, yhar = [p.reshape(2 * N1, DHH) for p in projs[:5]]
    sup, sdn = projs[5], projs[6]

    upr = up_idx[0].reshape(NB8, SUB)
    dnr = dn_idx[0].reshape(NB8, SUB)
    r_cat2 = jnp.stack([upr, dnr])
    upc = up_idx[1].reshape(NB8, SUB)
    dnc = dn_idx[1].reshape(NB8, SUB)
    c_cat2 = jnp.stack([upc, dnc])
    v_cat2 = jnp.stack([up_val.reshape(NB8, SUB), dn_val.reshape(NB8, SUB)])
    su_cat = jnp.stack([sup.reshape(2 * N1), sdn.reshape(2 * N1)])
    zeros = jnp.zeros((SPAD,), f32)

    w2, sv2 = _attn(su_cat, r_cat2, c_cat2, v_cat2, zeros)
    w_up, w_dn = w2[0], w2[1]
    sv_up, sv_dn = sv2[0], sv2[1]

    upc2 = jnp.stack([upc, upc + N1])
    dnc2 = jnp.stack([dnc, dnc + N1])

    iu = _spmm8(hup, yup1, upr, upc2, w_up)
    idn = _spmm8(hdn, ydn1, dnr, dnc2, w_dn)

    catr = jnp.concatenate([upr, dnr], axis=0)
    catc = jnp.concatenate([upc2, dnc2], axis=1)
    catv = jnp.concatenate([sv_up, sv_dn], axis=0)
    y = yhar
    for _ in range(ORDER):
        y = _spmm16(y, y, catr, catc, catv)

    t = _spmm8(y, iu, upr, upc2, w_up)
    z = _spmm8(t, idn, dnr, dnc2, w_dn)

    gid3 = x_bel_1.reshape(NBLK, 1, RB)
    return _readout(z.reshape(2, N1, DHH), gid3, W_out,
                    b_out.reshape(1, DOUT), W_ro, b_ro.reshape(1, DOUT))


# speedup vs baseline: 11.6693x; 1.4420x over previous
import functools

import jax
import jax.numpy as jnp
from jax import lax
from jax.experimental import pallas as pl
from jax.experimental.pallas import tpu as pltpu
from jax.experimental.pallas import tpu_sc as plsc

N1 = 50000
DIN, DB, DH, DOUT, NG = 128, 64, 64, 64, 64
DHH = DH // 2
NNZ = 800000
EPS = 0.1
ORDER = 5
NT = 16
SUB = 80
CB = 25
RT = N1 // NT
SPAD = 16 * 3136
RB = 2000
NBLK = N1 // RB

_MESH = plsc.VectorSubcoreMesh(core_axis_name="c", subcore_axis_name="s")
_SC_PARAMS = pltpu.CompilerParams(use_tc_tiling_on_sc=False,
                                  needs_layout_passes=False)


def _proj_body(x1_ref, Win_ref, bin_ref, Wcat_ref, Aup_ref, Adn_ref,
               hup, yup1, hdn, ydn1, yhar, sup_ref, sdn_ref):
    t = x1_ref[...] @ Win_ref[...] + bin_ref[...]
    p = t @ Wcat_ref[...]

    def split(m):
        return jnp.stack([m[:, :DHH], m[:, DHH:]])

    hup[...] = split(p[:, 0:64])
    yup1[...] = split(p[:, 64:128])
    hdn[...] = split(p[:, 128:192])
    ydn1[...] = split(p[:, 192:256])
    yhar[...] = split(p[:, 256:320])
    sup_ref[...] = p[:, 0:64] @ Aup_ref[...]
    sdn_ref[...] = p[:, 128:192] @ Adn_ref[...]


def _proj(x_1, W_in1, b_in1, Wcat, Aup, Adn):
    half = jax.ShapeDtypeStruct((2, N1, DHH), jnp.float32)
    svec = jax.ShapeDtypeStruct((N1, 2), jnp.float32)
    full_spec = lambda s: pl.BlockSpec(s, lambda i: (0, 0))
    return pl.pallas_call(
        _proj_body,
        grid=(NBLK,),
        in_specs=[pl.BlockSpec((RB, DIN), lambda i: (i, 0)),
                  full_spec((DIN, DB)), full_spec((1, DB)),
                  full_spec((DB, 320)), full_spec((DB, 2)), full_spec((DB, 2))],
        out_specs=[pl.BlockSpec((2, RB, DHH), lambda i: (0, i, 0))] * 5 +
                  [pl.BlockSpec((RB, 2), lambda i: (i, 0))] * 2,
        out_shape=[half] * 5 + [svec] * 2,
    )(x_1, W_in1, b_in1, Wcat, Aup, Adn)


_CRA = NNZ // SUB // NT


def _attn_body(su_cat, r_cat, c_cat, v_cat, zeros_hbm,
               w_out, sv_out,
               su_buf, rb, cb, vb, wb, svb, tb, ssb, ssum):
    cid = lax.axis_index("c")
    sid = lax.axis_index("s")
    r_hbm = r_cat.at[cid]
    c_hbm = c_cat.at[cid]
    v_hbm = v_cat.at[cid]
    w_hbm = w_out.at[cid]
    sv_hbm = sv_out.at[cid]

    def compute_t(k):
        def vec(j, _):
            sl = pl.ds(j * 16, 16)
            rv = rb[k, sl]
            cv = cb[k, sl]
            a = plsc.load_gather(su_buf, [rv * 2])
            b = plsc.load_gather(su_buf, [cv * 2 + 1])
            e = a + b
            e = jnp.where(e >= 0, e, e * jnp.float32(0.2))
            tb[sl] = jnp.exp(e)
            return 0
        lax.fori_loop(0, SUB // 16, vec, 0)

    pltpu.sync_copy(zeros_hbm.at[pl.ds(0, SPAD // NT)],
                    ssum.at[pl.ds(sid * (SPAD // NT), SPAD // NT)])
    pltpu.sync_copy(su_cat.at[cid], su_buf)
    plsc.subcore_barrier()
    cr0 = sid * _CRA

    def outer1(ob, _):
        cbase = cr0 + ob * CB
        pltpu.sync_copy(r_hbm.at[pl.ds(cbase, CB)], rb)
        pltpu.sync_copy(c_hbm.at[pl.ds(cbase, CB)], cb)

        def chunk(k, _):
            compute_t(k)
            pltpu.sync_copy(tb, ssum.at[rb.at[k]], add=True)
            return 0
        lax.fori_loop(0, CB, chunk, 0)
        return 0
    lax.fori_loop(0, _CRA // CB, outer1, 0)
    plsc.subcore_barrier()

    def outer2(ob, _):
        cbase = cr0 + ob * CB
        pltpu.sync_copy(r_hbm.at[pl.ds(cbase, CB)], rb)
        pltpu.sync_copy(c_hbm.at[pl.ds(cbase, CB)], cb)
        pltpu.sync_copy(v_hbm.at[pl.ds(cbase, CB)], vb)

        def chunk(k, _):
            compute_t(k)
            pltpu.sync_copy(ssum.at[rb.at[k]], ssb)

            def vec2(j, _):
                sl = pl.ds(j * 16, 16)
                v = vb[k, sl]
                wb[k, sl] = tb[sl] / (ssb[sl] + jnp.float32(1e-9)) * v
                svb[k, sl] = v * jnp.float32(-EPS)
                return 0
            lax.fori_loop(0, SUB // 16, vec2, 0)
            return 0
        lax.fori_loop(0, CB, chunk, 0)
        pltpu.sync_copy(wb, w_hbm.at[pl.ds(cbase, CB)])
        pltpu.sync_copy(svb, sv_hbm.at[pl.ds(cbase, CB)])
        return 0
    lax.fori_loop(0, _CRA // CB, outer2, 0)


_attn = functools.partial(
    pl.kernel,
    mesh=_MESH,
    compiler_params=_SC_PARAMS,
    out_type=[jax.ShapeDtypeStruct((2, NNZ // SUB, SUB), jnp.float32)] * 2,
    scratch_types=[
        pltpu.VMEM((2 * N1,), jnp.float32),
        pltpu.VMEM((CB, SUB), jnp.int32),
        pltpu.VMEM((CB, SUB), jnp.int32),
        pltpu.VMEM((CB, SUB), jnp.float32),
        pltpu.VMEM((CB, SUB), jnp.float32),
        pltpu.VMEM((CB, SUB), jnp.float32),
        pltpu.VMEM((SUB,), jnp.float32),
        pltpu.VMEM((SUB,), jnp.float32),
        pltpu.VMEM_SHARED((SPAD,), jnp.float32),
    ],
)(_attn_body)


CBS = 25
NPAIR = (CBS - 1) // 2


def _spmm_body(nchunks, init_h, y_h, r_hbm, c_cat, w_hbm, z_h,
               rb, cb, wb, G0, G1, sg0, sg1, ss0, ss1, zacc):
    cid = lax.axis_index("c")
    sid = lax.axis_index("s")
    cr_t = nchunks // NT
    base = cid * N1 + sid * RT

    pltpu.sync_copy(init_h.at[pl.ds(base, RT)], zacc.at[pl.ds(sid * RT, RT)])
    plsc.subcore_barrier()
    cr0 = sid * cr_t
    c_hbm = c_cat.at[cid]

    def scale(Gr, k):
        for g in range(SUB // 16):
            wv = wb[k, pl.ds(g * 16, 16)]
            for e in range(16):
                row = g * 16 + e
                w = jnp.broadcast_to(wv[e], (16,))
                Gr[row, pl.ds(0, 16)] = Gr[row, pl.ds(0, 16)] * w
                Gr[row, pl.ds(16, 16)] = Gr[row, pl.ds(16, 16)] * w

    def outer(ob, _):
        cbase = cr0 + ob * CBS
        pltpu.sync_copy(r_hbm.at[pl.ds(cbase, CBS)], rb)
        pltpu.sync_copy(c_hbm.at[pl.ds(cbase, CBS)], cb)
        pltpu.sync_copy(w_hbm.at[pl.ds(cbase, CBS)], wb)
        pltpu.async_copy(y_h.at[cb.at[0]], G0, sg0)
        pltpu.async_copy(y_h.at[cb.at[1]], G1, sg1)

        def pair(p, _):
            k0 = 2 * p
            k1 = k0 + 1
            pltpu.make_async_copy(y_h.at[cb.at[k0]], G0, sg0).wait()
            scale(G0, k0)
            pltpu.async_copy(G0, zacc.at[rb.at[k0]], ss0, add=True)
            pltpu.make_async_copy(y_h.at[cb.at[k1]], G1, sg1).wait()
            scale(G1, k1)
            pltpu.async_copy(G1, zacc.at[rb.at[k1]], ss1, add=True)
            pltpu.make_async_copy(G0, zacc.at[rb.at[k0]], ss0).wait()
            pltpu.async_copy(y_h.at[cb.at[k0 + 2]], G0, sg0)
            pltpu.make_async_copy(G1, zacc.at[rb.at[k1]], ss1).wait()

            @pl.when(p + 1 < NPAIR)
            def _():
                pltpu.async_copy(y_h.at[cb.at[k1 + 2]], G1, sg1)
            return 0
        lax.fori_loop(0, NPAIR, pair, 0)
        kt = CBS - 1
        pltpu.make_async_copy(y_h.at[cb.at[kt]], G0, sg0).wait()
        scale(G0, kt)
        pltpu.sync_copy(G0, zacc.at[rb.at[kt]], add=True)
        return 0
    lax.fori_loop(0, cr_t // CBS, outer, 0)
    plsc.subcore_barrier()
    pltpu.sync_copy(zacc.at[pl.ds(sid * RT, RT)], z_h.at[pl.ds(base, RT)])


def _make_spmm(nchunks):
    return functools.partial(
        pl.kernel,
        mesh=_MESH,
        compiler_params=_SC_PARAMS,
        out_type=jax.ShapeDtypeStruct((2 * N1, DHH), jnp.float32),
        scratch_types=[
            pltpu.VMEM((CBS, SUB), jnp.int32),
            pltpu.VMEM((CBS, SUB), jnp.int32),
            pltpu.VMEM((CBS, SUB), jnp.float32),
            pltpu.VMEM((SUB, DHH), jnp.float32),
            pltpu.VMEM((SUB, DHH), jnp.float32),
            pltpu.SemaphoreType.DMA,
            pltpu.SemaphoreType.DMA,
            pltpu.SemaphoreType.DMA,
            pltpu.SemaphoreType.DMA,
            pltpu.VMEM_SHARED((N1, DHH), jnp.float32),
        ],
    )(functools.partial(_spmm_body, nchunks))


_spmm8 = _make_spmm(NNZ // SUB)
_spmm16 = _make_spmm(2 * NNZ // SUB)


def _ro_body(z_ref, gid_ref, Wout_ref, bout_ref, Wro_ref, bro_ref,
             out_ref, acc_ref):
    i = pl.program_id(0)

    @pl.when(i == 0)
    def _():
        acc_ref[...] = jnp.zeros((NG, DOUT), jnp.float32)

    z = jnp.concatenate([z_ref[0], z_ref[1]], axis=1)
    x = jnp.maximum(z, 0.0) @ Wout_ref[...] + bout_ref[...]
    g = gid_ref[0, 0, :]
    oh = (g[:, None] == lax.broadcasted_iota(jnp.int32, (RB, NG), 1))
    oh = oh.astype(jnp.float32)
    acc_ref[...] += lax.dot_general(oh, x, (((0,), (0,)), ((), ())))

    @pl.when(i == pl.num_programs(0) - 1)
    def _():
        out_ref[...] = acc_ref[...] @ Wro_ref[...] + bro_ref[...]


def _readout(z2, gid3, W_out, b_out, W_ro, b_ro):
    full_spec = lambda s: pl.BlockSpec(s, lambda i: tuple(0 for _ in s))
    return pl.pallas_call(
        _ro_body,
        grid=(NBLK,),
        in_specs=[pl.BlockSpec((2, RB, DHH), lambda i: (0, i, 0)),
                  pl.BlockSpec((1, 1, RB), lambda i: (i, 0, 0)),
                  full_spec((DH, DOUT)), full_spec((1, DOUT)),
                  full_spec((DOUT, DOUT)), full_spec((1, DOUT))],
        out_specs=full_spec((NG, DOUT)),
        out_shape=jax.ShapeDtypeStruct((NG, DOUT), jnp.float32),
        scratch_shapes=[pltpu.VMEM((NG, DOUT), jnp.float32)],
    )(z2, gid3, W_out, b_out, W_ro, b_ro)


def kernel(x_0, x_1, x_2, up_idx, up_val, dn_idx, dn_val, inc_idx, inc_val,
           x_bel_1, W_in0, b_in0, W_in1, b_in1, W_in2, b_in2,
           W_up, a_up, W_dn, a_dn, W_har, W_out, b_out, W_ro, b_ro):
    f32 = jnp.float32
    NB8 = NNZ // SUB
    Wcat = jnp.concatenate([W_up[0], W_up[1], W_dn[0], W_dn[1], W_har], axis=1)
    Aup = a_up.reshape(2, DH).T
    Adn = a_dn.reshape(2, DH).T

    projs = _proj(x_1, W_in1, b_in1.reshape(1, DB), Wcat, Aup, Adn)
    hup, yup1, hdn, ydn1, yhar = [p.reshape(2 * N1, DHH) for p in projs[:5]]
    sup, sdn = projs[5], projs[6]

    upr = up_idx[0].reshape(NB8, SUB)
    dnr = dn_idx[0].reshape(NB8, SUB)
    r_cat2 = jnp.stack([upr, dnr])
    upc = up_idx[1].reshape(NB8, SUB)
    dnc = dn_idx[1].reshape(NB8, SUB)
    c_cat2 = jnp.stack([upc, dnc])
    v_cat2 = jnp.stack([up_val.reshape(NB8, SUB), dn_val.reshape(NB8, SUB)])
    su_cat = jnp.stack([sup.reshape(2 * N1), sdn.reshape(2 * N1)])
    zeros = jnp.zeros((SPAD,), f32)

    w2, sv2 = _attn(su_cat, r_cat2, c_cat2, v_cat2, zeros)
    w_up, w_dn = w2[0], w2[1]
    sv_up, sv_dn = sv2[0], sv2[1]

    upc2 = jnp.stack([upc, upc + N1])
    dnc2 = jnp.stack([dnc, dnc + N1])

    iu = _spmm8(hup, yup1, upr, upc2, w_up)
    idn = _spmm8(hdn, ydn1, dnr, dnc2, w_dn)

    catr = jnp.concatenate([upr, dnr], axis=0)
    catc = jnp.concatenate([upc2, dnc2], axis=1)
    catv = jnp.concatenate([sv_up, sv_dn], axis=0)
    y = yhar
    for _ in range(ORDER):
        y = _spmm16(y, y, catr, catc, catv)

    t = _spmm8(y, iu, upr, upc2, w_up)
    z = _spmm8(t, idn, dnr, dnc2, w_dn)

    gid3 = x_bel_1.reshape(NBLK, 1, RB)
    return _readout(z.reshape(2, N1, DHH), gid3, W_out,
                    b_out.reshape(1, DOUT), W_ro, b_ro.reshape(1, DOUT))


# speedup vs baseline: 11.8607x; 1.0164x over previous
import functools

import jax
import jax.numpy as jnp
from jax import lax
from jax.experimental import pallas as pl
from jax.experimental.pallas import tpu as pltpu
from jax.experimental.pallas import tpu_sc as plsc

N1 = 50000
DIN, DB, DH, DOUT, NG = 128, 64, 64, 64, 64
DHH = DH // 2
NNZ = 800000
EPS = 0.1
ORDER = 5
NT = 16
SUB = 80
CB = 25
RT = N1 // NT
SPAD = 16 * 3136
RB = 2000
NBLK = N1 // RB

_MESH = plsc.VectorSubcoreMesh(core_axis_name="c", subcore_axis_name="s")
_SC_PARAMS = pltpu.CompilerParams(use_tc_tiling_on_sc=False,
                                  needs_layout_passes=False)


def _proj_body(x1_ref, Win_ref, bin_ref, Wcat_ref, Aup_ref, Adn_ref,
               hup, yup1, hdn, ydn1, yhar, sup_ref, sdn_ref):
    t = x1_ref[...] @ Win_ref[...] + bin_ref[...]
    p = t @ Wcat_ref[...]

    def split(m):
        return jnp.stack([m[:, :DHH], m[:, DHH:]])

    hup[...] = split(p[:, 0:64])
    yup1[...] = split(p[:, 64:128])
    hdn[...] = split(p[:, 128:192])
    ydn1[...] = split(p[:, 192:256])
    yhar[...] = split(p[:, 256:320])
    sup_ref[...] = p[:, 0:64] @ Aup_ref[...]
    sdn_ref[...] = p[:, 128:192] @ Adn_ref[...]


def _proj(x_1, W_in1, b_in1, Wcat, Aup, Adn):
    half = jax.ShapeDtypeStruct((2, N1, DHH), jnp.float32)
    svec = jax.ShapeDtypeStruct((N1, 2), jnp.float32)
    full_spec = lambda s: pl.BlockSpec(s, lambda i: (0, 0))
    return pl.pallas_call(
        _proj_body,
        grid=(NBLK,),
        in_specs=[pl.BlockSpec((RB, DIN), lambda i: (i, 0)),
                  full_spec((DIN, DB)), full_spec((1, DB)),
                  full_spec((DB, 320)), full_spec((DB, 2)), full_spec((DB, 2))],
        out_specs=[pl.BlockSpec((2, RB, DHH), lambda i: (0, i, 0))] * 5 +
                  [pl.BlockSpec((RB, 2), lambda i: (i, 0))] * 2,
        out_shape=[half] * 5 + [svec] * 2,
    )(x_1, W_in1, b_in1, Wcat, Aup, Adn)


_CRA = NNZ // SUB // NT


_NPA = (CB - 1) // 2


def _attn_body(su_cat, r_cat, c_cat, v_cat, zeros_hbm,
               w_out, sv_out,
               su_buf, rb, cb, vb, wb, svb, tb0, tb1, ssb0, ssb1,
               sa0, sa1, sb0, sb1, ssum):
    cid = lax.axis_index("c")
    sid = lax.axis_index("s")
    r_hbm = r_cat.at[cid]
    c_hbm = c_cat.at[cid]
    v_hbm = v_cat.at[cid]
    w_hbm = w_out.at[cid]
    sv_hbm = sv_out.at[cid]

    def compute_t(k, tref):
        def vec(j, _):
            sl = pl.ds(j * 16, 16)
            rv = rb[k, sl]
            cv = cb[k, sl]
            a = plsc.load_gather(su_buf, [rv * 2])
            b = plsc.load_gather(su_buf, [cv * 2 + 1])
            e = a + b
            e = jnp.where(e >= 0, e, e * jnp.float32(0.2))
            tref[sl] = jnp.exp(e)
            return 0
        lax.fori_loop(0, SUB // 16, vec, 0)

    pltpu.sync_copy(zeros_hbm.at[pl.ds(0, SPAD // NT)],
                    ssum.at[pl.ds(sid * (SPAD // NT), SPAD // NT)])
    pltpu.sync_copy(su_cat.at[cid], su_buf)
    plsc.subcore_barrier()
    cr0 = sid * _CRA

    def outer1(ob, _):
        cbase = cr0 + ob * CB
        pltpu.sync_copy(r_hbm.at[pl.ds(cbase, CB)], rb)
        pltpu.sync_copy(c_hbm.at[pl.ds(cbase, CB)], cb)

        def pair(p, _):
            k0 = 2 * p
            k1 = k0 + 1
            compute_t(k0, tb0)
            pltpu.async_copy(tb0, ssum.at[rb.at[k0]], sa0, add=True)
            compute_t(k1, tb1)
            pltpu.async_copy(tb1, ssum.at[rb.at[k1]], sa1, add=True)
            pltpu.make_async_copy(tb0, ssum.at[rb.at[k0]], sa0).wait()
            pltpu.make_async_copy(tb1, ssum.at[rb.at[k1]], sa1).wait()
            return 0
        lax.fori_loop(0, _NPA, pair, 0)
        compute_t(CB - 1, tb0)
        pltpu.sync_copy(tb0, ssum.at[rb.at[CB - 1]], add=True)
        return 0
    lax.fori_loop(0, _CRA // CB, outer1, 0)
    plsc.subcore_barrier()

    def outer2(ob, _):
        cbase = cr0 + ob * CB
        pltpu.sync_copy(r_hbm.at[pl.ds(cbase, CB)], rb)
        pltpu.sync_copy(c_hbm.at[pl.ds(cbase, CB)], cb)
        pltpu.sync_copy(v_hbm.at[pl.ds(cbase, CB)], vb)

        def vec2(k, tref, ssref):
            def body(j, _):
                sl = pl.ds(j * 16, 16)
                v = vb[k, sl]
                wb[k, sl] = tref[sl] / (ssref[sl] + jnp.float32(1e-9)) * v
                svb[k, sl] = v * jnp.float32(-EPS)
                return 0
            lax.fori_loop(0, SUB // 16, body, 0)

        pltpu.async_copy(ssum.at[rb.at[0]], ssb0, sb0)
        pltpu.async_copy(ssum.at[rb.at[1]], ssb1, sb1)

        def pair(p, _):
            k0 = 2 * p
            k1 = k0 + 1
            compute_t(k0, tb0)
            pltpu.make_async_copy(ssum.at[rb.at[k0]], ssb0, sb0).wait()
            vec2(k0, tb0, ssb0)
            pltpu.async_copy(ssum.at[rb.at[k0 + 2]], ssb0, sb0)
            compute_t(k1, tb1)
            pltpu.make_async_copy(ssum.at[rb.at[k1]], ssb1, sb1).wait()
            vec2(k1, tb1, ssb1)

            @pl.when(p + 1 < _NPA)
            def _():
                pltpu.async_copy(ssum.at[rb.at[k1 + 2]], ssb1, sb1)
            return 0
        lax.fori_loop(0, _NPA, pair, 0)
        kt = CB - 1
        compute_t(kt, tb0)
        pltpu.make_async_copy(ssum.at[rb.at[kt]], ssb0, sb0).wait()
        vec2(kt, tb0, ssb0)
        pltpu.sync_copy(wb, w_hbm.at[pl.ds(cbase, CB)])
        pltpu.sync_copy(svb, sv_hbm.at[pl.ds(cbase, CB)])
        return 0
    lax.fori_loop(0, _CRA // CB, outer2, 0)


_attn = functools.partial(
    pl.kernel,
    mesh=_MESH,
    compiler_params=_SC_PARAMS,
    out_type=[jax.ShapeDtypeStruct((2, NNZ // SUB, SUB), jnp.float32)] * 2,
    scratch_types=[
        pltpu.VMEM((2 * N1,), jnp.float32),
        pltpu.VMEM((CB, SUB), jnp.int32),
        pltpu.VMEM((CB, SUB), jnp.int32),
        pltpu.VMEM((CB, SUB), jnp.float32),
        pltpu.VMEM((CB, SUB), jnp.float32),
        pltpu.VMEM((CB, SUB), jnp.float32),
        pltpu.VMEM((SUB,), jnp.float32),
        pltpu.VMEM((SUB,), jnp.float32),
        pltpu.VMEM((SUB,), jnp.float32),
        pltpu.VMEM((SUB,), jnp.float32),
        pltpu.SemaphoreType.DMA,
        pltpu.SemaphoreType.DMA,
        pltpu.SemaphoreType.DMA,
        pltpu.SemaphoreType.DMA,
        pltpu.VMEM_SHARED((SPAD,), jnp.float32),
    ],
)(_attn_body)


CBS = 25
NPAIR = (CBS - 1) // 2


def _spmm_body(nchunks, init_h, y_h, r_hbm, c_cat, w_hbm, z_h,
               rb, cb, wb, G0, G1, sg0, sg1, ss0, ss1, zacc):
    cid = lax.axis_index("c")
    sid = lax.axis_index("s")
    cr_t = nchunks // NT
    base = cid * N1 + sid * RT

    pltpu.sync_copy(init_h.at[pl.ds(base, RT)], zacc.at[pl.ds(sid * RT, RT)])
    plsc.subcore_barrier()
    cr0 = sid * cr_t
    c_hbm = c_cat.at[cid]

    def scale(Gr, k):
        for g in range(SUB // 16):
            wv = wb[k, pl.ds(g * 16, 16)]
            for e in range(16):
                row = g * 16 + e
                w = jnp.broadcast_to(wv[e], (16,))
                Gr[row, pl.ds(0, 16)] = Gr[row, pl.ds(0, 16)] * w
                Gr[row, pl.ds(16, 16)] = Gr[row, pl.ds(16, 16)] * w

    def outer(ob, _):
        cbase = cr0 + ob * CBS
        pltpu.sync_copy(r_hbm.at[pl.ds(cbase, CBS)], rb)
        pltpu.sync_copy(c_hbm.at[pl.ds(cbase, CBS)], cb)
        pltpu.sync_copy(w_hbm.at[pl.ds(cbase, CBS)], wb)
        pltpu.async_copy(y_h.at[cb.at[0]], G0, sg0)
        pltpu.async_copy(y_h.at[cb.at[1]], G1, sg1)

        def pair(p, _):
            k0 = 2 * p
            k1 = k0 + 1
            pltpu.make_async_copy(y_h.at[cb.at[k0]], G0, sg0).wait()
            scale(G0, k0)
            pltpu.async_copy(G0, zacc.at[rb.at[k0]], ss0, add=True)
            pltpu.make_async_copy(y_h.at[cb.at[k1]], G1, sg1).wait()
            scale(G1, k1)
            pltpu.async_copy(G1, zacc.at[rb.at[k1]], ss1, add=True)
            pltpu.make_async_copy(G0, zacc.at[rb.at[k0]], ss0).wait()
            pltpu.async_copy(y_h.at[cb.at[k0 + 2]], G0, sg0)
            pltpu.make_async_copy(G1, zacc.at[rb.at[k1]], ss1).wait()

            @pl.when(p + 1 < NPAIR)
            def _():
                pltpu.async_copy(y_h.at[cb.at[k1 + 2]], G1, sg1)
            return 0
        lax.fori_loop(0, NPAIR, pair, 0)
        kt = CBS - 1
        pltpu.make_async_copy(y_h.at[cb.at[kt]], G0, sg0).wait()
        scale(G0, kt)
        pltpu.sync_copy(G0, zacc.at[rb.at[kt]], add=True)
        return 0
    lax.fori_loop(0, cr_t // CBS, outer, 0)
    plsc.subcore_barrier()
    pltpu.sync_copy(zacc.at[pl.ds(sid * RT, RT)], z_h.at[pl.ds(base, RT)])


def _make_spmm(nchunks):
    return functools.partial(
        pl.kernel,
        mesh=_MESH,
        compiler_params=_SC_PARAMS,
        out_type=jax.ShapeDtypeStruct((2 * N1, DHH), jnp.float32),
        scratch_types=[
            pltpu.VMEM((CBS, SUB), jnp.int32),
            pltpu.VMEM((CBS, SUB), jnp.int32),
            pltpu.VMEM((CBS, SUB), jnp.float32),
            pltpu.VMEM((SUB, DHH), jnp.float32),
            pltpu.VMEM((SUB, DHH), jnp.float32),
            pltpu.SemaphoreType.DMA,
            pltpu.SemaphoreType.DMA,
            pltpu.SemaphoreType.DMA,
            pltpu.SemaphoreType.DMA,
            pltpu.VMEM_SHARED((N1, DHH), jnp.float32),
        ],
    )(functools.partial(_spmm_body, nchunks))


_spmm8 = _make_spmm(NNZ // SUB)
_spmm16 = _make_spmm(2 * NNZ // SUB)


def _ro_body(z_ref, gid_ref, Wout_ref, bout_ref, Wro_ref, bro_ref,
             out_ref, acc_ref):
    i = pl.program_id(0)

    @pl.when(i == 0)
    def _():
        acc_ref[...] = jnp.zeros((NG, DOUT), jnp.float32)

    z = jnp.concatenate([z_ref[0], z_ref[1]], axis=1)
    x = jnp.maximum(z, 0.0) @ Wout_ref[...] + bout_ref[...]
    g = gid_ref[0, 0, :]
    oh = (g[:, None] == lax.broadcasted_iota(jnp.int32, (RB, NG), 1))
    oh = oh.astype(jnp.float32)
    acc_ref[...] += lax.dot_general(oh, x, (((0,), (0,)), ((), ())))

    @pl.when(i == pl.num_programs(0) - 1)
    def _():
        out_ref[...] = acc_ref[...] @ Wro_ref[...] + bro_ref[...]


def _readout(z2, gid3, W_out, b_out, W_ro, b_ro):
    full_spec = lambda s: pl.BlockSpec(s, lambda i: tuple(0 for _ in s))
    return pl.pallas_call(
        _ro_body,
        grid=(NBLK,),
        in_specs=[pl.BlockSpec((2, RB, DHH), lambda i: (0, i, 0)),
                  pl.BlockSpec((1, 1, RB), lambda i: (i, 0, 0)),
                  full_spec((DH, DOUT)), full_spec((1, DOUT)),
                  full_spec((DOUT, DOUT)), full_spec((1, DOUT))],
        out_specs=full_spec((NG, DOUT)),
        out_shape=jax.ShapeDtypeStruct((NG, DOUT), jnp.float32),
        scratch_shapes=[pltpu.VMEM((NG, DOUT), jnp.float32)],
    )(z2, gid3, W_out, b_out, W_ro, b_ro)


def kernel(x_0, x_1, x_2, up_idx, up_val, dn_idx, dn_val, inc_idx, inc_val,
           x_bel_1, W_in0, b_in0, W_in1, b_in1, W_in2, b_in2,
           W_up, a_up, W_dn, a_dn, W_har, W_out, b_out, W_ro, b_ro):
    f32 = jnp.float32
    NB8 = NNZ // SUB
    Wcat = jnp.concatenate([W_up[0], W_up[1], W_dn[0], W_dn[1], W_har], axis=1)
    Aup = a_up.reshape(2, DH).T
    Adn = a_dn.reshape(2, DH).T

    projs = _proj(x_1, W_in1, b_in1.reshape(1, DB), Wcat, Aup, Adn)
    hup, yup1, hdn, ydn1, yhar = [p.reshape(2 * N1, DHH) for p in projs[:5]]
    sup, sdn = projs[5], projs[6]

    upr = up_idx[0].reshape(NB8, SUB)
    dnr = dn_idx[0].reshape(NB8, SUB)
    r_cat2 = jnp.stack([upr, dnr])
    upc = up_idx[1].reshape(NB8, SUB)
    dnc = dn_idx[1].reshape(NB8, SUB)
    c_cat2 = jnp.stack([upc, dnc])
    v_cat2 = jnp.stack([up_val.reshape(NB8, SUB), dn_val.reshape(NB8, SUB)])
    su_cat = jnp.stack([sup.reshape(2 * N1), sdn.reshape(2 * N1)])
    zeros = jnp.zeros((SPAD,), f32)

    w2, sv2 = _attn(su_cat, r_cat2, c_cat2, v_cat2, zeros)
    w_up, w_dn = w2[0], w2[1]
    sv_up, sv_dn = sv2[0], sv2[1]

    upc2 = jnp.stack([upc, upc + N1])
    dnc2 = jnp.stack([dnc, dnc + N1])

    iu = _spmm8(hup, yup1, upr, upc2, w_up)
    idn = _spmm8(hdn, ydn1, dnr, dnc2, w_dn)

    catr = jnp.concatenate([upr, dnr], axis=0)
    catc = jnp.concatenate([upc2, dnc2], axis=1)
    catv = jnp.concatenate([sv_up, sv_dn], axis=0)
    y = yhar
    for _ in range(ORDER):
        y = _spmm16(y, y, catr, catc, catv)

    t = _spmm8(y, iu, upr, upc2, w_up)
    z = _spmm8(t, idn, dnr, dnc2, w_dn)

    gid3 = x_bel_1.reshape(NBLK, 1, RB)
    return _readout(z.reshape(2, N1, DHH), gid3, W_out,
                    b_out.reshape(1, DOUT), W_ro, b_ro.reshape(1, DOUT))


# speedup vs baseline: 16.6732x; 1.4058x over previous
import functools

import jax
import jax.numpy as jnp
from jax import lax
from jax.experimental import pallas as pl
from jax.experimental.pallas import tpu as pltpu
from jax.experimental.pallas import tpu_sc as plsc

N1 = 50000
DIN, DB, DH, DOUT, NG = 128, 64, 64, 64, 64
DHH = DH // 2
NNZ = 800000
EPS = 0.1
ORDER = 5
NT = 16
SUB = 80
CB = 25
RT = N1 // NT
SPAD = 16 * 3136
RB = 2000
NBLK = N1 // RB

_MESH = plsc.VectorSubcoreMesh(core_axis_name="c", subcore_axis_name="s")
_SC_PARAMS = pltpu.CompilerParams(use_tc_tiling_on_sc=False,
                                  needs_layout_passes=False)


def _proj_body(x1_ref, Win_ref, bin_ref, Wcat_ref, Aup_ref, Adn_ref,
               hup, yup1, hdn, ydn1, yhar, sup_ref, sdn_ref):
    t = x1_ref[...] @ Win_ref[...] + bin_ref[...]
    p = t @ Wcat_ref[...]

    def split(m):
        return jnp.stack([m[:, :DHH], m[:, DHH:]])

    hup[...] = split(p[:, 0:64])
    yup1[...] = split(p[:, 64:128])
    hdn[...] = split(p[:, 128:192])
    ydn1[...] = split(p[:, 192:256])
    yhar[...] = split(p[:, 256:320])
    sup_ref[...] = p[:, 0:64] @ Aup_ref[...]
    sdn_ref[...] = p[:, 128:192] @ Adn_ref[...]


def _proj(x_1, W_in1, b_in1, Wcat, Aup, Adn):
    half = jax.ShapeDtypeStruct((2, N1, DHH), jnp.float32)
    svec = jax.ShapeDtypeStruct((N1, 2), jnp.float32)
    full_spec = lambda s: pl.BlockSpec(s, lambda i: (0, 0))
    return pl.pallas_call(
        _proj_body,
        grid=(NBLK,),
        in_specs=[pl.BlockSpec((RB, DIN), lambda i: (i, 0)),
                  full_spec((DIN, DB)), full_spec((1, DB)),
                  full_spec((DB, 320)), full_spec((DB, 2)), full_spec((DB, 2))],
        out_specs=[pl.BlockSpec((2, RB, DHH), lambda i: (0, i, 0))] * 5 +
                  [pl.BlockSpec((RB, 2), lambda i: (i, 0))] * 2,
        out_shape=[half] * 5 + [svec] * 2,
    )(x_1, W_in1, b_in1, Wcat, Aup, Adn)


_CRA = NNZ // SUB // NT


_NPA = (CB - 1) // 2


def _attn_body(su_cat, r_cat, c_cat, v_cat, zeros_hbm,
               w_out, sv_out,
               su_buf, rb, cb, vb, wb, svb, tb0, tb1, ssb0, ssb1,
               sa0, sa1, sb0, sb1, ssum):
    cid = lax.axis_index("c")
    sid = lax.axis_index("s")
    r_hbm = r_cat.at[cid]
    c_hbm = c_cat.at[cid]
    v_hbm = v_cat.at[cid]
    w_hbm = w_out.at[cid]
    sv_hbm = sv_out.at[cid]

    def compute_t(k, tref):
        def vec(j, _):
            sl = pl.ds(j * 16, 16)
            rv = rb[k, sl]
            cv = cb[k, sl]
            a = plsc.load_gather(su_buf, [rv * 2])
            b = plsc.load_gather(su_buf, [cv * 2 + 1])
            e = a + b
            e = jnp.where(e >= 0, e, e * jnp.float32(0.2))
            tref[sl] = jnp.exp(e)
            return 0
        lax.fori_loop(0, SUB // 16, vec, 0)

    pltpu.sync_copy(zeros_hbm.at[pl.ds(0, SPAD // NT)],
                    ssum.at[pl.ds(sid * (SPAD // NT), SPAD // NT)])
    pltpu.sync_copy(su_cat.at[cid], su_buf)
    plsc.subcore_barrier()
    cr0 = sid * _CRA

    def outer1(ob, _):
        cbase = cr0 + ob * CB
        pltpu.sync_copy(r_hbm.at[pl.ds(cbase, CB)], rb)
        pltpu.sync_copy(c_hbm.at[pl.ds(cbase, CB)], cb)

        def pair(p, _):
            k0 = 2 * p
            k1 = k0 + 1
            compute_t(k0, tb0)
            pltpu.async_copy(tb0, ssum.at[rb.at[k0]], sa0, add=True)
            compute_t(k1, tb1)
            pltpu.async_copy(tb1, ssum.at[rb.at[k1]], sa1, add=True)
            pltpu.make_async_copy(tb0, ssum.at[rb.at[k0]], sa0).wait()
            pltpu.make_async_copy(tb1, ssum.at[rb.at[k1]], sa1).wait()
            return 0
        lax.fori_loop(0, _NPA, pair, 0)
        compute_t(CB - 1, tb0)
        pltpu.sync_copy(tb0, ssum.at[rb.at[CB - 1]], add=True)
        return 0
    lax.fori_loop(0, _CRA // CB, outer1, 0)
    plsc.subcore_barrier()

    def outer2(ob, _):
        cbase = cr0 + ob * CB
        pltpu.sync_copy(r_hbm.at[pl.ds(cbase, CB)], rb)
        pltpu.sync_copy(c_hbm.at[pl.ds(cbase, CB)], cb)
        pltpu.sync_copy(v_hbm.at[pl.ds(cbase, CB)], vb)

        def vec2(k, tref, ssref):
            def body(j, _):
                sl = pl.ds(j * 16, 16)
                v = vb[k, sl]
                wb[k, sl] = tref[sl] / (ssref[sl] + jnp.float32(1e-9)) * v
                svb[k, sl] = v * jnp.float32(-EPS)
                return 0
            lax.fori_loop(0, SUB // 16, body, 0)

        pltpu.async_copy(ssum.at[rb.at[0]], ssb0, sb0)
        pltpu.async_copy(ssum.at[rb.at[1]], ssb1, sb1)

        def pair(p, _):
            k0 = 2 * p
            k1 = k0 + 1
            compute_t(k0, tb0)
            pltpu.make_async_copy(ssum.at[rb.at[k0]], ssb0, sb0).wait()
            vec2(k0, tb0, ssb0)
            pltpu.async_copy(ssum.at[rb.at[k0 + 2]], ssb0, sb0)
            compute_t(k1, tb1)
            pltpu.make_async_copy(ssum.at[rb.at[k1]], ssb1, sb1).wait()
            vec2(k1, tb1, ssb1)

            @pl.when(p + 1 < _NPA)
            def _():
                pltpu.async_copy(ssum.at[rb.at[k1 + 2]], ssb1, sb1)
            return 0
        lax.fori_loop(0, _NPA, pair, 0)
        kt = CB - 1
        compute_t(kt, tb0)
        pltpu.make_async_copy(ssum.at[rb.at[kt]], ssb0, sb0).wait()
        vec2(kt, tb0, ssb0)
        pltpu.sync_copy(wb, w_hbm.at[pl.ds(cbase, CB)])
        pltpu.sync_copy(svb, sv_hbm.at[pl.ds(cbase, CB)])
        return 0
    lax.fori_loop(0, _CRA // CB, outer2, 0)


_attn = functools.partial(
    pl.kernel,
    mesh=_MESH,
    compiler_params=_SC_PARAMS,
    out_type=[jax.ShapeDtypeStruct((2, NNZ // SUB, SUB), jnp.float32)] * 2,
    scratch_types=[
        pltpu.VMEM((2 * N1,), jnp.float32),
        pltpu.VMEM((CB, SUB), jnp.int32),
        pltpu.VMEM((CB, SUB), jnp.int32),
        pltpu.VMEM((CB, SUB), jnp.float32),
        pltpu.VMEM((CB, SUB), jnp.float32),
        pltpu.VMEM((CB, SUB), jnp.float32),
        pltpu.VMEM((SUB,), jnp.float32),
        pltpu.VMEM((SUB,), jnp.float32),
        pltpu.VMEM((SUB,), jnp.float32),
        pltpu.VMEM((SUB,), jnp.float32),
        pltpu.SemaphoreType.DMA,
        pltpu.SemaphoreType.DMA,
        pltpu.SemaphoreType.DMA,
        pltpu.SemaphoreType.DMA,
        pltpu.VMEM_SHARED((SPAD,), jnp.float32),
    ],
)(_attn_body)


CBS = 25
NPAIR = (CBS - 1) // 2


NQ = (CBS - 1) // 4


def _spmm_body(nchunks, init_h, y_h, r_hbm, c_cat, w_hbm, z_h,
               rb, cb, wb, G0, G1, G2, G3,
               sg0, sg1, sg2, sg3, ss0, ss1, ss2, ss3, zacc):
    cid = lax.axis_index("c")
    sid = lax.axis_index("s")
    cr_t = nchunks // NT
    base = cid * N1 + sid * RT

    pltpu.sync_copy(init_h.at[pl.ds(base, RT)], zacc.at[pl.ds(sid * RT, RT)])
    plsc.subcore_barrier()
    cr0 = sid * cr_t
    c_hbm = c_cat.at[cid]
    Gs = (G0, G1, G2, G3)
    sg = (sg0, sg1, sg2, sg3)
    ss = (ss0, ss1, ss2, ss3)

    def scale(Gr, k):
        for g in range(SUB // 16):
            wv = wb[k, pl.ds(g * 16, 16)]
            for e in range(16):
                row = g * 16 + e
                w = jnp.broadcast_to(wv[e], (16,))
                Gr[row, pl.ds(0, 16)] = Gr[row, pl.ds(0, 16)] * w
                Gr[row, pl.ds(16, 16)] = Gr[row, pl.ds(16, 16)] * w

    def slot(k, i):
        b = Gs[i]
        b2 = Gs[(i + 2) % 4]

        @pl.when(k >= 2)
        def _():
            pltpu.make_async_copy(
                b2, zacc.at[rb.at[jnp.maximum(k - 2, 0)]], ss[(i + 2) % 4]).wait()

        @pl.when(k + 2 < CBS)
        def _():
            pltpu.async_copy(
                y_h.at[cb.at[jnp.minimum(k + 2, CBS - 1)]], b2, sg[(i + 2) % 4])
        pltpu.make_async_copy(y_h.at[cb.at[k]], b, sg[i]).wait()
        scale(b, k)
        pltpu.async_copy(b, zacc.at[rb.at[k]], ss[i], add=True)

    def outer(ob, _):
        cbase = cr0 + ob * CBS
        pltpu.sync_copy(r_hbm.at[pl.ds(cbase, CBS)], rb)
        pltpu.sync_copy(c_hbm.at[pl.ds(cbase, CBS)], cb)
        pltpu.sync_copy(w_hbm.at[pl.ds(cbase, CBS)], wb)
        pltpu.async_copy(y_h.at[cb.at[0]], G0, sg0)
        pltpu.async_copy(y_h.at[cb.at[1]], G1, sg1)

        def quad(q, _):
            k0 = 4 * q
            for i in range(4):
                slot(k0 + i, i)
            return 0
        lax.fori_loop(0, NQ, quad, 0)
        slot(CBS - 1, 0)
        pltpu.make_async_copy(G3, zacc.at[rb.at[0]], ss3).wait()
        pltpu.make_async_copy(G0, zacc.at[rb.at[0]], ss0).wait()
        return 0
    lax.fori_loop(0, cr_t // CBS, outer, 0)
    plsc.subcore_barrier()
    pltpu.sync_copy(zacc.at[pl.ds(sid * RT, RT)], z_h.at[pl.ds(base, RT)])


def _make_spmm(nchunks):
    return functools.partial(
        pl.kernel,
        mesh=_MESH,
        compiler_params=_SC_PARAMS,
        out_type=jax.ShapeDtypeStruct((2 * N1, DHH), jnp.float32),
        scratch_types=[
            pltpu.VMEM((CBS, SUB), jnp.int32),
            pltpu.VMEM((CBS, SUB), jnp.int32),
            pltpu.VMEM((CBS, SUB), jnp.float32),
            pltpu.VMEM((SUB, DHH), jnp.float32),
            pltpu.VMEM((SUB, DHH), jnp.float32),
            pltpu.VMEM((SUB, DHH), jnp.float32),
            pltpu.VMEM((SUB, DHH), jnp.float32),
            pltpu.SemaphoreType.DMA,
            pltpu.SemaphoreType.DMA,
            pltpu.SemaphoreType.DMA,
            pltpu.SemaphoreType.DMA,
            pltpu.SemaphoreType.DMA,
            pltpu.SemaphoreType.DMA,
            pltpu.SemaphoreType.DMA,
            pltpu.SemaphoreType.DMA,
            pltpu.VMEM_SHARED((N1, DHH), jnp.float32),
        ],
    )(functools.partial(_spmm_body, nchunks))


_spmm8 = _make_spmm(NNZ // SUB)
_spmm16 = _make_spmm(2 * NNZ // SUB)


def _ro_body(z_ref, gid_ref, Wout_ref, bout_ref, Wro_ref, bro_ref,
             out_ref, acc_ref):
    i = pl.program_id(0)

    @pl.when(i == 0)
    def _():
        acc_ref[...] = jnp.zeros((NG, DOUT), jnp.float32)

    z = jnp.concatenate([z_ref[0], z_ref[1]], axis=1)
    x = jnp.maximum(z, 0.0) @ Wout_ref[...] + bout_ref[...]
    g = gid_ref[0, 0, :]
    oh = (g[:, None] == lax.broadcasted_iota(jnp.int32, (RB, NG), 1))
    oh = oh.astype(jnp.float32)
    acc_ref[...] += lax.dot_general(oh, x, (((0,), (0,)), ((), ())))

    @pl.when(i == pl.num_programs(0) - 1)
    def _():
        out_ref[...] = acc_ref[...] @ Wro_ref[...] + bro_ref[...]


def _readout(z2, gid3, W_out, b_out, W_ro, b_ro):
    full_spec = lambda s: pl.BlockSpec(s, lambda i: tuple(0 for _ in s))
    return pl.pallas_call(
        _ro_body,
        grid=(NBLK,),
        in_specs=[pl.BlockSpec((2, RB, DHH), lambda i: (0, i, 0)),
                  pl.BlockSpec((1, 1, RB), lambda i: (i, 0, 0)),
                  full_spec((DH, DOUT)), full_spec((1, DOUT)),
                  full_spec((DOUT, DOUT)), full_spec((1, DOUT))],
        out_specs=full_spec((NG, DOUT)),
        out_shape=jax.ShapeDtypeStruct((NG, DOUT), jnp.float32),
        scratch_shapes=[pltpu.VMEM((NG, DOUT), jnp.float32)],
    )(z2, gid3, W_out, b_out, W_ro, b_ro)


def kernel(x_0, x_1, x_2, up_idx, up_val, dn_idx, dn_val, inc_idx, inc_val,
           x_bel_1, W_in0, b_in0, W_in1, b_in1, W_in2, b_in2,
           W_up, a_up, W_dn, a_dn, W_har, W_out, b_out, W_ro, b_ro):
    f32 = jnp.float32
    NB8 = NNZ // SUB
    Wcat = jnp.concatenate([W_up[0], W_up[1], W_dn[0], W_dn[1], W_har], axis=1)
    Aup = a_up.reshape(2, DH).T
    Adn = a_dn.reshape(2, DH).T

    projs = _proj(x_1, W_in1, b_in1.reshape(1, DB), Wcat, Aup, Adn)
    hup, yup1, hdn, ydn1, yhar = [p.reshape(2 * N1, DHH) for p in projs[:5]]
    sup, sdn = projs[5], projs[6]

    upr = up_idx[0].reshape(NB8, SUB)
    dnr = dn_idx[0].reshape(NB8, SUB)
    r_cat2 = jnp.stack([upr, dnr])
    upc = up_idx[1].reshape(NB8, SUB)
    dnc = dn_idx[1].reshape(NB8, SUB)
    c_cat2 = jnp.stack([upc, dnc])
    v_cat2 = jnp.stack([up_val.reshape(NB8, SUB), dn_val.reshape(NB8, SUB)])
    su_cat = jnp.stack([sup.reshape(2 * N1), sdn.reshape(2 * N1)])
    zeros = jnp.zeros((SPAD,), f32)

    w2, sv2 = _attn(su_cat, r_cat2, c_cat2, v_cat2, zeros)
    w_up, w_dn = w2[0], w2[1]
    sv_up, sv_dn = sv2[0], sv2[1]

    upc2 = jnp.stack([upc, upc + N1])
    dnc2 = jnp.stack([dnc, dnc + N1])

    iu = _spmm8(hup, yup1, upr, upc2, w_up)
    idn = _spmm8(hdn, ydn1, dnr, dnc2, w_dn)

    catr = jnp.concatenate([upr, dnr], axis=0)
    catc = jnp.concatenate([upc2, dnc2], axis=1)
    catv = jnp.concatenate([sv_up, sv_dn], axis=0)
    y = yhar
    for _ in range(ORDER):
        y = _spmm16(y, y, catr, catc, catv)

    t = _spmm8(y, iu, upr, upc2, w_up)
    z = _spmm8(t, idn, dnr, dnc2, w_dn)

    gid3 = x_bel_1.reshape(NBLK, 1, RB)
    return _readout(z.reshape(2, N1, DHH), gid3, W_out,
                    b_out.reshape(1, DOUT), W_ro, b_ro.reshape(1, DOUT))
